# baseline (speedup 1.0000x reference)
import functools

import jax
import jax.numpy as jnp
from jax import lax
from jax.experimental import pallas as pl
from jax.experimental.pallas import tpu as pltpu
from jax.experimental.pallas import tpu_sc as plsc

_L = 16


def _sc_pe_add(x2, pe, row0=0, nb=None):
    BS, D = x2.shape
    S = pe.shape[0]
    NC, NS = 2, 16
    NW = NC * NS
    NB = (BS // S) if nb is None else nb
    P = S // NW
    R = 8 if NB > 2 else 32
    NCI = P // R
    NV = D // _L
    mesh = plsc.VectorSubcoreMesh(core_axis_name="c", subcore_axis_name="s")

    @functools.partial(
        pl.kernel,
        mesh=mesh,
        out_type=jax.ShapeDtypeStruct((NB * S, D), jnp.float32),
        scratch_types=(
            [pltpu.VMEM((2, R, D), jnp.float32)]
            + [pltpu.VMEM((2, NB, R, D), jnp.float32)]
            + [pltpu.SemaphoreType.DMA] * 2
            + [pltpu.SemaphoreType.DMA] * (2 * NB)
            + [pltpu.SemaphoreType.DMA] * (2 * NB)
        ),
    )
    def k(x_hbm, pe_hbm, out_hbm, pe_v, xb, *sems):
        sem_pe = sems[0:2]
        sem_in = sems[2:2 + 2 * NB]
        sem_out = sems[2 + 2 * NB:2 + 4 * NB]
        wid = lax.axis_index("s") * NC + lax.axis_index("c")
        pe_base = wid * P

        def pe_copy(ci, q):
            return pltpu.make_async_copy(
                pe_hbm.at[pl.ds(pe_base + ci * R, R)], pe_v.at[q], sem_pe[q])

        def in_copy(ci, b, q):
            row = row0 + b * S + pe_base + ci * R
            return pltpu.make_async_copy(
                x_hbm.at[pl.ds(row, R)], xb.at[q, b], sem_in[q * NB + b])

        def out_copy(ci, b, q):
            row = b * S + pe_base + ci * R
            return pltpu.make_async_copy(
                xb.at[q, b], out_hbm.at[pl.ds(row, R)], sem_out[q * NB + b])

        pe_copy(0, 0).start()
        for b in range(NB):
            in_copy(0, b, 0).start()

        def group(g, carry):
            for q in (0, 1):
                ci = g * 2 + q
                nq = 1 - q

                @pl.when(ci + 1 < NCI)
                def _issue():
                    pe_copy(ci + 1, nq).start()

                pe_copy(ci, q).wait()
                for b in range(NB):
                    in_copy(ci, b, q).wait()

                    def row_body(r, c3, b=b):
                        for j in range(NV):
                            sl = pl.ds(j * _L, _L)
                            plsc.addupdate(xb.at[q, b, r, sl], pe_v[q, r, sl])
                        return c3

                    lax.fori_loop(0, R, row_body, 0, unroll=False)
                    out_copy(ci, b, q).start()

                    @pl.when(ci + 1 < NCI)
                    def _issue_b(b=b):
                        @pl.when(ci >= 1)
                        def _wait_prev_out():
                            out_copy(ci - 1, b, nq).wait()
                        in_copy(ci + 1, b, nq).start()
            return carry

        lax.fori_loop(0, NCI // 2, group, 0, unroll=False)

        for b in range(NB):
            out_copy(NCI - 2, b, 0).wait()
            out_copy(NCI - 1, b, 1).wait()

    return k(x2, pe)


def _tc_add_body(x_ref, pe_ref, o_ref):
    o_ref[...] = x_ref[...] + pe_ref[...]


def kernel(x, pe):
    B, S, D = x.shape
    BSPLIT = B - 1

    ST = 2048
    tc_out = pl.pallas_call(
        _tc_add_body,
        grid=(S // ST, BSPLIT),
        in_specs=[
            pl.BlockSpec((1, ST, D), lambda i, b: (b, i, 0)),
            pl.BlockSpec((ST, D), lambda i, b: (i, 0)),
        ],
        out_specs=pl.BlockSpec((1, ST, D), lambda i, b: (b, i, 0)),
        out_shape=jax.ShapeDtypeStruct((B, S, D), x.dtype),
    )(x, pe)

    sc_out = _sc_pe_add(x.reshape(B * S, D), pe, row0=BSPLIT * S, nb=B - BSPLIT)

    def _merge_body(_, sc_ref, o_ref):
        o_ref[...] = sc_ref[...][None]

    return pl.pallas_call(
        _merge_body,
        grid=(B - BSPLIT, S // ST),
        in_specs=[
            pl.BlockSpec((1, ST, D), lambda b, i: (BSPLIT + b, i, 0)),
            pl.BlockSpec((ST, D), lambda b, i: (b * (S // ST) + i, 0)),
        ],
        out_specs=pl.BlockSpec((1, ST, D), lambda b, i: (BSPLIT + b, i, 0)),
        out_shape=jax.ShapeDtypeStruct((B, S, D), x.dtype),
        input_output_aliases={0: 0},
    )(tc_out, sc_out)


# speedup vs baseline: 1.2969x; 1.2969x over previous
import functools

import jax
import jax.numpy as jnp
from jax import lax
from jax.experimental import pallas as pl
from jax.experimental.pallas import tpu as pltpu
from jax.experimental.pallas import tpu_sc as plsc

_L = 16


def _sc_pe_add(x2, pe, row0=0, nb=None):
    BS, D = x2.shape
    S = pe.shape[0]
    NC, NS = 2, 16
    NW = NC * NS
    NB = (BS // S) if nb is None else nb
    P = S // NW
    R = 8 if NB > 2 else 32
    NCI = P // R
    NV = D // _L
    mesh = plsc.VectorSubcoreMesh(core_axis_name="c", subcore_axis_name="s")

    @functools.partial(
        pl.kernel,
        mesh=mesh,
        out_type=jax.ShapeDtypeStruct((NB * S, D), jnp.float32),
        scratch_types=(
            [pltpu.VMEM((2, R, D), jnp.float32)]
            + [pltpu.VMEM((2, NB, R, D), jnp.float32)]
            + [pltpu.SemaphoreType.DMA] * 2
            + [pltpu.SemaphoreType.DMA] * (2 * NB)
            + [pltpu.SemaphoreType.DMA] * (2 * NB)
        ),
    )
    def k(x_hbm, pe_hbm, out_hbm, pe_v, xb, *sems):
        sem_pe = sems[0:2]
        sem_in = sems[2:2 + 2 * NB]
        sem_out = sems[2 + 2 * NB:2 + 4 * NB]
        wid = lax.axis_index("s") * NC + lax.axis_index("c")
        pe_base = wid * P

        def pe_copy(ci, q):
            return pltpu.make_async_copy(
                pe_hbm.at[pl.ds(pe_base + ci * R, R)], pe_v.at[q], sem_pe[q])

        def in_copy(ci, b, q):
            row = row0 + b * S + pe_base + ci * R
            return pltpu.make_async_copy(
                x_hbm.at[pl.ds(row, R)], xb.at[q, b], sem_in[q * NB + b])

        def out_copy(ci, b, q):
            row = b * S + pe_base + ci * R
            return pltpu.make_async_copy(
                xb.at[q, b], out_hbm.at[pl.ds(row, R)], sem_out[q * NB + b])

        pe_copy(0, 0).start()
        for b in range(NB):
            in_copy(0, b, 0).start()

        def group(g, carry):
            for q in (0, 1):
                ci = g * 2 + q
                nq = 1 - q

                @pl.when(ci + 1 < NCI)
                def _issue():
                    pe_copy(ci + 1, nq).start()

                pe_copy(ci, q).wait()
                for b in range(NB):
                    in_copy(ci, b, q).wait()

                    def row_body(r, c3, b=b):
                        for j in range(NV):
                            sl = pl.ds(j * _L, _L)
                            plsc.addupdate(xb.at[q, b, r, sl], pe_v[q, r, sl])
                        return c3

                    lax.fori_loop(0, R, row_body, 0, unroll=False)
                    out_copy(ci, b, q).start()

                    @pl.when(ci + 1 < NCI)
                    def _issue_b(b=b):
                        @pl.when(ci >= 1)
                        def _wait_prev_out():
                            out_copy(ci - 1, b, nq).wait()
                        in_copy(ci + 1, b, nq).start()
            return carry

        lax.fori_loop(0, NCI // 2, group, 0, unroll=False)

        for b in range(NB):
            out_copy(NCI - 2, b, 0).wait()
            out_copy(NCI - 1, b, 1).wait()

    return k(x2, pe)


def kernel(x, pe):
    B, S, D = x.shape
    return _sc_pe_add(x.reshape(B * S, D), pe).reshape(B, S, D)
